# baseline (speedup 1.0000x reference)
import functools

import jax
import jax.numpy as jnp
from jax import lax
from jax.experimental import pallas as pl
from jax.experimental.pallas import tpu as pltpu
from jax.experimental.pallas import tpu_sc as plsc

N = 10000
C = 20
G = 128
E = 320000
H = 8

DW = 32
NW = 32
EPT = E // NW
PG = 2

NP = 10240
BN = 1024
GRID = NP // BN


def _dense_body(x_ref, mu_ref, var_ref, w_ref, s_ref, llw_ref, lrw_ref,
                llb_ref, lrb_ref, p_ref, src_ref, dst_ref, ll_ref):
    i = pl.program_id(0)
    wb = w_ref[...]
    m = jnp.max(wb, axis=1, keepdims=True)
    ew = jnp.exp(wb - m)
    p = ew / jnp.sum(ew, axis=1, keepdims=True)
    lq = jnp.log(p + 1e-8)

    mu = mu_ref[...]
    iv = 1.0 / var_ref[...]
    mv = mu * iv
    dvec = jnp.sum(mu * mv, axis=1)

    xb = x_ref[...]
    a = lax.dot_general(xb * xb, iv, (((1,), (1,)), ((), ())),
                        preferred_element_type=jnp.float32)
    b = lax.dot_general(xb, mv, (((1,), (1,)), ((), ())),
                        preferred_element_type=jnp.float32)
    s = s_ref[...]
    f = -0.5 * (a - 2.0 * s * b + (s * s) * dvec[None, :])
    rowid = i * BN + lax.broadcasted_iota(jnp.int32, (BN, 1), 0)
    part = jnp.sum(jnp.where(rowid < N, p * f, 0.0)) * (1.0 / N)

    @pl.when(i == 0)
    def _():
        ll_ref[...] = jnp.zeros((1, 1), jnp.float32)

    ll_ref[...] += jnp.reshape(part, (1, 1))

    xl_t = lax.dot_general(llw_ref[...], xb, (((1,), (1,)), ((), ())),
                           preferred_element_type=jnp.float32) + llb_ref[...]
    xr_t = lax.dot_general(lrw_ref[...], xb, (((1,), (1,)), ((), ())),
                           preferred_element_type=jnp.float32) + lrb_ref[...]
    p_t = lax.dot_general(jnp.eye(C, dtype=jnp.float32), p,
                          (((1,), (1,)), ((), ())),
                          preferred_element_type=jnp.float32)
    lq_t = jnp.log(p_t + 1e-8)
    zpad = jnp.zeros((DW - H - C, BN), jnp.float32)
    src_ref[...] = jnp.concatenate([xl_t, p_t, zpad], axis=0)
    dst_ref[...] = jnp.concatenate([xr_t, lq_t, zpad], axis=0)
    p_ref[...] = p


_dense_call = pl.pallas_call(
    _dense_body,
    grid=(GRID,),
    in_specs=[
        pl.BlockSpec((BN, G), lambda i: (i, 0)),
        pl.BlockSpec((C, G), lambda i: (0, 0)),
        pl.BlockSpec((C, G), lambda i: (0, 0)),
        pl.BlockSpec((BN, C), lambda i: (i, 0)),
        pl.BlockSpec((BN, 1), lambda i: (i, 0)),
        pl.BlockSpec((H, G), lambda i: (0, 0)),
        pl.BlockSpec((H, G), lambda i: (0, 0)),
        pl.BlockSpec((H, 1), lambda i: (0, 0)),
        pl.BlockSpec((H, 1), lambda i: (0, 0)),
    ],
    out_specs=[
        pl.BlockSpec((BN, C), lambda i: (i, 0)),
        pl.BlockSpec((DW, BN), lambda i: (0, i)),
        pl.BlockSpec((DW, BN), lambda i: (0, i)),
        pl.BlockSpec((1, 1), lambda i: (0, 0)),
    ],
    out_shape=[
        jax.ShapeDtypeStruct((NP, C), jnp.float32),
        jax.ShapeDtypeStruct((DW, NP), jnp.float32),
        jax.ShapeDtypeStruct((DW, NP), jnp.float32),
        jax.ShapeDtypeStruct((1, 1), jnp.float32),
    ],
)


_sc_mesh = plsc.VectorSubcoreMesh(core_axis_name="c", subcore_axis_name="s")


@functools.partial(
    pl.kernel,
    mesh=_sc_mesh,
    compiler_params=pltpu.CompilerParams(
        needs_layout_passes=False, use_tc_tiling_on_sc=False),
    out_type=(
        jax.ShapeDtypeStruct((NW, N), jnp.float32),
        jax.ShapeDtypeStruct((NW, N), jnp.float32),
    ),
    scratch_types=[
        pltpu.VMEM((EPT,), jnp.int32),
        pltpu.VMEM((EPT,), jnp.int32),
        pltpu.VMEM((PG, NP), jnp.float32),
        pltpu.VMEM((PG, NP), jnp.float32),
        pltpu.VMEM((EPT,), jnp.float32),
        pltpu.VMEM((N,), jnp.float32),
        pltpu.VMEM((N,), jnp.float32),
        pltpu.VMEM((16,), jnp.float32),
    ],
)
def _edge_kernel(src_tab, dst_tab, eidx, att16, den_out, t_out,
                 src_idx, dst_idx, buf_a, buf_b, w_arr, acc_d, acc_t, att_v):
    cid = lax.axis_index("c")
    sid = lax.axis_index("s")
    wid = sid * 2 + cid
    ebase = wid * EPT
    ngrp = EPT // 16

    pltpu.sync_copy(eidx.at[0, pl.ds(ebase, EPT)], src_idx)
    pltpu.sync_copy(eidx.at[1, pl.ds(ebase, EPT)], dst_idx)
    pltpu.sync_copy(att16, att_v)
    attvec = att_v[...]

    def _zero(k, carry):
        acc_d[pl.ds(k * 16, 16)] = jnp.zeros((16,), jnp.float32)
        acc_t[pl.ds(k * 16, 16)] = jnp.zeros((16,), jnp.float32)
        return carry

    lax.fori_loop(0, N // 16, _zero, 0)

    for j0 in range(0, H, PG):
        pltpu.sync_copy(src_tab.at[pl.ds(j0, PG)], buf_a)
        pltpu.sync_copy(dst_tab.at[pl.ds(j0, PG)], buf_b)
        att_p = [attvec[j0 + p] for p in range(PG)]
        first = j0 == 0

        def _sbody(k, carry, att_p=att_p, first=first):
            off = k * 16
            srcv = src_idx[pl.ds(off, 16)]
            dstv = dst_idx[pl.ds(off, 16)]
            sa = jnp.zeros((16,), jnp.float32)
            for p in range(PG):
                rowv = jnp.full((16,), p, jnp.int32)
                av = plsc.load_gather(buf_a, [rowv, srcv])
                bv = plsc.load_gather(buf_b, [rowv, dstv])
                z = av + bv
                z = jnp.where(z >= 0.0, z, 0.2 * z)
                sa = sa + att_p[p] * z
            if not first:
                sa = sa + w_arr[pl.ds(off, 16)]
            w_arr[pl.ds(off, 16)] = sa
            return carry

        lax.fori_loop(0, ngrp, _sbody, 0)

    def _wbody(k, carry):
        off = k * 16
        wv = jnp.exp(w_arr[pl.ds(off, 16)])
        w_arr[pl.ds(off, 16)] = wv
        plsc.addupdate_scatter(acc_d, [dst_idx[pl.ds(off, 16)]], wv)
        return carry

    lax.fori_loop(0, ngrp, _wbody, 0)

    for j0 in range(H, H + C, PG):
        pltpu.sync_copy(src_tab.at[pl.ds(j0, PG)], buf_a)
        pltpu.sync_copy(dst_tab.at[pl.ds(j0, PG)], buf_b)

        def _qbody(k, carry):
            off = k * 16
            srcv = src_idx[pl.ds(off, 16)]
            dstv = dst_idx[pl.ds(off, 16)]
            qc = jnp.zeros((16,), jnp.float32)
            for p in range(PG):
                rowv = jnp.full((16,), p, jnp.int32)
                av = plsc.load_gather(buf_a, [rowv, srcv])
                bv = plsc.load_gather(buf_b, [rowv, dstv])
                qc = qc + av * bv
            plsc.addupdate_scatter(acc_t, [dstv], w_arr[pl.ds(off, 16)] * qc)
            return carry

        lax.fori_loop(0, ngrp, _qbody, 0)

    pltpu.sync_copy(acc_d, den_out.at[wid])
    pltpu.sync_copy(acc_t, t_out.at[wid])


def _fin_body(d_ref, t_ref, ce_ref):
    d = jnp.sum(d_ref[...], axis=0)
    t = jnp.sum(t_ref[...], axis=0)
    safe = jnp.where(d > 0.0, d, 1.0)
    ce = -jnp.sum(jnp.where(d > 0.0, t / safe, 0.0)) * (1.0 / N)
    ce_ref[...] = jnp.reshape(ce, (1, 1))


_fin_call = pl.pallas_call(
    _fin_body,
    in_specs=[
        pl.BlockSpec((NW, N), lambda: (0, 0)),
        pl.BlockSpec((NW, N), lambda: (0, 0)),
    ],
    out_specs=pl.BlockSpec((1, 1), lambda: (0, 0)),
    out_shape=jax.ShapeDtypeStruct((1, 1), jnp.float32),
)


def kernel(X, Mu, Var, W, S, lin_l_w, lin_l_b, lin_r_w, lin_r_b, att, edge_index):
    pad = ((0, NP - N), (0, 0))
    p, src_tab, dst_tab, ll = _dense_call(
        jnp.pad(X, pad), Mu, Var, jnp.pad(W, pad), jnp.pad(S, pad),
        lin_l_w, lin_r_w, lin_l_b.reshape(H, 1), lin_r_b.reshape(H, 1))
    att16 = jnp.pad(att, (0, 16 - H))
    den, t = _edge_kernel(src_tab, dst_tab, edge_index, att16)
    ce = _fin_call(den, t)
    return (ll[0, 0], ce[0, 0], p[:N])


# speedup vs baseline: 1.2397x; 1.2397x over previous
import functools

import jax
import jax.numpy as jnp
from jax import lax
from jax.experimental import pallas as pl
from jax.experimental.pallas import tpu as pltpu
from jax.experimental.pallas import tpu_sc as plsc

N = 10000
C = 20
G = 128
E = 320000
H = 8

DW = 32
PW = DW // 2
NW = 32
EPT = E // NW
EB = 80
NCHUNK = EPT // EB
GPC = EB // 16

BN = 1000
GRID = N // BN


def _dense_body(x_ref, mu_ref, var_ref, w_ref, s_ref, llw_ref, lrw_ref,
                llb_ref, lrb_ref, p_ref, src_ref, dst_ref, ll_ref):
    i = pl.program_id(0)
    wb = w_ref[...]
    m = jnp.max(wb, axis=1, keepdims=True)
    ew = jnp.exp(wb - m)
    p = ew / jnp.sum(ew, axis=1, keepdims=True)
    lq = jnp.log(p + 1e-8)

    mu = mu_ref[...]
    iv = 1.0 / var_ref[...]
    mv = mu * iv
    dvec = jnp.sum(mu * mv, axis=1)

    xb = x_ref[...]
    a = lax.dot_general(xb * xb, iv, (((1,), (1,)), ((), ())),
                        preferred_element_type=jnp.float32)
    b = lax.dot_general(xb, mv, (((1,), (1,)), ((), ())),
                        preferred_element_type=jnp.float32)
    s = s_ref[...]
    f = -0.5 * (a - 2.0 * s * b + (s * s) * dvec[None, :])
    part = jnp.sum(p * f) * (1.0 / N)

    @pl.when(i == 0)
    def _():
        ll_ref[...] = jnp.zeros((1, 1), jnp.float32)

    ll_ref[...] += jnp.reshape(part, (1, 1))

    xl = lax.dot_general(xb, llw_ref[...], (((1,), (1,)), ((), ())),
                         preferred_element_type=jnp.float32) + llb_ref[...]
    xr = lax.dot_general(xb, lrw_ref[...], (((1,), (1,)), ((), ())),
                         preferred_element_type=jnp.float32) + lrb_ref[...]
    zpad = jnp.zeros((BN, DW - H - C), jnp.float32)
    src_ref[...] = jnp.concatenate([xl, p, zpad], axis=1)
    dst_ref[...] = jnp.concatenate([xr, lq, zpad], axis=1)
    p_ref[...] = p


_dense_call = pl.pallas_call(
    _dense_body,
    grid=(GRID,),
    in_specs=[
        pl.BlockSpec((BN, G), lambda i: (i, 0)),
        pl.BlockSpec((C, G), lambda i: (0, 0)),
        pl.BlockSpec((C, G), lambda i: (0, 0)),
        pl.BlockSpec((BN, C), lambda i: (i, 0)),
        pl.BlockSpec((BN, 1), lambda i: (i, 0)),
        pl.BlockSpec((H, G), lambda i: (0, 0)),
        pl.BlockSpec((H, G), lambda i: (0, 0)),
        pl.BlockSpec((1, H), lambda i: (0, 0)),
        pl.BlockSpec((1, H), lambda i: (0, 0)),
    ],
    out_specs=[
        pl.BlockSpec((BN, C), lambda i: (i, 0)),
        pl.BlockSpec((BN, DW), lambda i: (i, 0)),
        pl.BlockSpec((BN, DW), lambda i: (i, 0)),
        pl.BlockSpec((1, 1), lambda i: (0, 0)),
    ],
    out_shape=[
        jax.ShapeDtypeStruct((N, C), jnp.float32),
        jax.ShapeDtypeStruct((N, DW), jnp.float32),
        jax.ShapeDtypeStruct((N, DW), jnp.float32),
        jax.ShapeDtypeStruct((1, 1), jnp.float32),
    ],
)


_sc_mesh = plsc.VectorSubcoreMesh(core_axis_name="c", subcore_axis_name="s")


@functools.partial(
    pl.kernel,
    mesh=_sc_mesh,
    compiler_params=pltpu.CompilerParams(
        needs_layout_passes=False, use_tc_tiling_on_sc=False),
    out_type=(
        jax.ShapeDtypeStruct((NW, N), jnp.float32),
        jax.ShapeDtypeStruct((NW, N), jnp.float32),
    ),
    scratch_types=[
        pltpu.VMEM((EPT,), jnp.int32),
        pltpu.VMEM((EPT,), jnp.int32),
        pltpu.VMEM((EB, PW), jnp.int32),
        pltpu.VMEM((EB, PW), jnp.int32),
        pltpu.VMEM((EB, PW), jnp.int32),
        pltpu.VMEM((EB, PW), jnp.int32),
        pltpu.VMEM((N,), jnp.float32),
        pltpu.VMEM((N,), jnp.float32),
        pltpu.VMEM((128,), jnp.float32),
        pltpu.SemaphoreType.DMA,
        pltpu.SemaphoreType.DMA,
    ],
)
def _edge_kernel(src_tab, dst_tab, eidx, att_re, den_out, t_out,
                 src_idx, dst_idx, bs0, bs1, bd0, bd1, acc_d, acc_t,
                 att_v, sem0, sem1):
    cid = lax.axis_index("c")
    sid = lax.axis_index("s")
    wid = sid * 2 + cid
    ebase = wid * EPT

    pltpu.sync_copy(eidx.at[0, pl.ds(ebase, EPT)], src_idx)
    pltpu.sync_copy(eidx.at[1, pl.ds(ebase, EPT)], dst_idx)
    pltpu.sync_copy(att_re, att_v)

    def _zero(k, carry):
        acc_d[pl.ds(k * 16, 16)] = jnp.zeros((16,), jnp.float32)
        acc_t[pl.ds(k * 16, 16)] = jnp.zeros((16,), jnp.float32)
        return carry

    lax.fori_loop(0, N // 16, _zero, 0)

    iota16 = lax.iota(jnp.int32, 16)
    ae = [att_v[pl.ds(j * 16, 16)] for j in range(4)]
    ao = [att_v[pl.ds(64 + j * 16, 16)] for j in range(4)]
    w4 = [jnp.bitwise_and(iota16 + j, 3) for j in range(4)]
    w10 = []
    for j in range(10):
        cj = iota16 + j
        cj = jnp.where(cj >= 10, cj - 10, cj)
        w10.append(cj + 4)
    himask = jnp.full((16,), -65536, jnp.int32)

    def unpk(v):
        lo = plsc.bitcast(lax.shift_left(v, 16), jnp.float32)
        hi = plsc.bitcast(jnp.bitwise_and(v, himask), jnp.float32)
        return lo, hi

    def fire(c, bs, bd, sem):
        pltpu.async_copy(src_tab.at[src_idx.at[pl.ds(c * EB, EB)]], bs, sem)
        pltpu.async_copy(dst_tab.at[dst_idx.at[pl.ds(c * EB, EB)]], bd, sem)

    def drain(bs, bd, sem):
        pltpu.make_async_copy(src_tab.at[pl.ds(0, EB)], bs, sem).wait()
        pltpu.make_async_copy(dst_tab.at[pl.ds(0, EB)], bd, sem).wait()

    def compute(c, bs, bd):
        for g in range(GPC):
            rows = iota16 + (g * 16)
            dstv = dst_idx[pl.ds(c * EB + g * 16, 16)]
            sa = jnp.zeros((16,), jnp.float32)
            for j in range(4):
                sv = plsc.load_gather(bs, [rows, w4[j]])
                dv = plsc.load_gather(bd, [rows, w4[j]])
                slo, shi = unpk(sv)
                dlo, dhi = unpk(dv)
                zlo = slo + dlo
                zlo = jnp.where(zlo >= 0.0, zlo, 0.2 * zlo)
                zhi = shi + dhi
                zhi = jnp.where(zhi >= 0.0, zhi, 0.2 * zhi)
                sa = sa + ae[j] * zlo + ao[j] * zhi
            qc = jnp.zeros((16,), jnp.float32)
            for j in range(10):
                sv = plsc.load_gather(bs, [rows, w10[j]])
                dv = plsc.load_gather(bd, [rows, w10[j]])
                slo, shi = unpk(sv)
                dlo, dhi = unpk(dv)
                qc = qc + slo * dlo + shi * dhi
            wv = jnp.exp(sa)
            plsc.addupdate_scatter(acc_d, [dstv], wv)
            plsc.addupdate_scatter(acc_t, [dstv], wv * qc)

    fire(0, bs0, bd0, sem0)

    def body(cc, carry):
        c0 = cc * 2
        drain(bs0, bd0, sem0)
        fire(c0 + 1, bs1, bd1, sem1)
        compute(c0, bs0, bd0)
        drain(bs1, bd1, sem1)
        fire(c0 + 2, bs0, bd0, sem0)
        compute(c0 + 1, bs1, bd1)
        return carry

    lax.fori_loop(0, (NCHUNK - 1) // 2, body, 0)

    drain(bs0, bd0, sem0)
    compute(NCHUNK - 1, bs0, bd0)

    pltpu.sync_copy(acc_d, den_out.at[wid])
    pltpu.sync_copy(acc_t, t_out.at[wid])


def _fin_body(d_ref, t_ref, ce_ref):
    d = jnp.sum(d_ref[...], axis=0)
    t = jnp.sum(t_ref[...], axis=0)
    safe = jnp.where(d > 0.0, d, 1.0)
    ce = -jnp.sum(jnp.where(d > 0.0, t / safe, 0.0)) * (1.0 / N)
    ce_ref[...] = jnp.reshape(ce, (1, 1))


_fin_call = pl.pallas_call(
    _fin_body,
    in_specs=[
        pl.BlockSpec((NW, N), lambda: (0, 0)),
        pl.BlockSpec((NW, N), lambda: (0, 0)),
    ],
    out_specs=pl.BlockSpec((1, 1), lambda: (0, 0)),
    out_shape=jax.ShapeDtypeStruct((1, 1), jnp.float32),
)


def kernel(X, Mu, Var, W, S, lin_l_w, lin_l_b, lin_r_w, lin_r_b, att, edge_index):
    p, src_tab, dst_tab, ll = _dense_call(
        X, Mu, Var, W, S, lin_l_w, lin_r_w,
        lin_l_b.reshape(1, H), lin_r_b.reshape(1, H))
    src_pk = lax.bitcast_convert_type(
        src_tab.astype(jnp.bfloat16).reshape(N, PW, 2), jnp.int32)
    dst_pk = lax.bitcast_convert_type(
        dst_tab.astype(jnp.bfloat16).reshape(N, PW, 2), jnp.int32)
    r = jnp.arange(16)
    att_re = jnp.concatenate(
        [att[2 * ((j + r) % 4)] for j in range(4)]
        + [att[2 * ((j + r) % 4) + 1] for j in range(4)])
    den, t = _edge_kernel(src_pk, dst_pk, edge_index, att_re)
    ce = _fin_call(den, t)
    return (ll[0, 0], ce[0, 0], p)


# speedup vs baseline: 1.4175x; 1.1434x over previous
import functools

import jax
import jax.numpy as jnp
from jax import lax
from jax.experimental import pallas as pl
from jax.experimental.pallas import tpu as pltpu
from jax.experimental.pallas import tpu_sc as plsc

N = 10000
C = 20
G = 128
E = 320000
H = 8

DW = 32
PW = DW // 2
NW = 32
EPT = E // NW
EB = 80
NCHUNK = EPT // EB
GPC = EB // 16

BN = 1000
GRID = N // BN


def _dense_body(x_ref, mu_ref, var_ref, w_ref, s_ref, llw_ref, lrw_ref,
                llb_ref, lrb_ref, p_ref, src_ref, dst_ref, ll_ref):
    i = pl.program_id(0)
    wb = w_ref[...]
    m = jnp.max(wb, axis=1, keepdims=True)
    ew = jnp.exp(wb - m)
    p = ew / jnp.sum(ew, axis=1, keepdims=True)
    lq = jnp.log(p + 1e-8)

    mu = mu_ref[...]
    iv = 1.0 / var_ref[...]
    mv = mu * iv
    dvec = jnp.sum(mu * mv, axis=1)

    xb = x_ref[...]
    a = lax.dot_general(xb * xb, iv, (((1,), (1,)), ((), ())),
                        preferred_element_type=jnp.float32)
    b = lax.dot_general(xb, mv, (((1,), (1,)), ((), ())),
                        preferred_element_type=jnp.float32)
    s = s_ref[...]
    f = -0.5 * (a - 2.0 * s * b + (s * s) * dvec[None, :])
    part = jnp.sum(p * f) * (1.0 / N)

    @pl.when(i == 0)
    def _():
        ll_ref[...] = jnp.zeros((1, 1), jnp.float32)

    ll_ref[...] += jnp.reshape(part, (1, 1))

    xl = lax.dot_general(xb, llw_ref[...], (((1,), (1,)), ((), ())),
                         preferred_element_type=jnp.float32) + llb_ref[...]
    xr = lax.dot_general(xb, lrw_ref[...], (((1,), (1,)), ((), ())),
                         preferred_element_type=jnp.float32) + lrb_ref[...]
    zpad = jnp.zeros((BN, DW - H - C), jnp.float32)
    src_ref[...] = jnp.concatenate([xl, p, zpad], axis=1)
    dst_ref[...] = jnp.concatenate([xr, lq, zpad], axis=1)
    p_ref[...] = p


_dense_call = pl.pallas_call(
    _dense_body,
    grid=(GRID,),
    in_specs=[
        pl.BlockSpec((BN, G), lambda i: (i, 0)),
        pl.BlockSpec((C, G), lambda i: (0, 0)),
        pl.BlockSpec((C, G), lambda i: (0, 0)),
        pl.BlockSpec((BN, C), lambda i: (i, 0)),
        pl.BlockSpec((BN, 1), lambda i: (i, 0)),
        pl.BlockSpec((H, G), lambda i: (0, 0)),
        pl.BlockSpec((H, G), lambda i: (0, 0)),
        pl.BlockSpec((1, H), lambda i: (0, 0)),
        pl.BlockSpec((1, H), lambda i: (0, 0)),
    ],
    out_specs=[
        pl.BlockSpec((BN, C), lambda i: (i, 0)),
        pl.BlockSpec((BN, DW), lambda i: (i, 0)),
        pl.BlockSpec((BN, DW), lambda i: (i, 0)),
        pl.BlockSpec((1, 1), lambda i: (0, 0)),
    ],
    out_shape=[
        jax.ShapeDtypeStruct((N, C), jnp.float32),
        jax.ShapeDtypeStruct((N, DW), jnp.float32),
        jax.ShapeDtypeStruct((N, DW), jnp.float32),
        jax.ShapeDtypeStruct((1, 1), jnp.float32),
    ],
)


_sc_mesh = plsc.VectorSubcoreMesh(core_axis_name="c", subcore_axis_name="s")


@functools.partial(
    pl.kernel,
    mesh=_sc_mesh,
    compiler_params=pltpu.CompilerParams(
        needs_layout_passes=False, use_tc_tiling_on_sc=False),
    out_type=(
        jax.ShapeDtypeStruct((NW, N), jnp.float32),
        jax.ShapeDtypeStruct((NW, N), jnp.float32),
    ),
    scratch_types=[
        pltpu.VMEM((EPT,), jnp.int32),
        pltpu.VMEM((EPT,), jnp.int32),
        pltpu.VMEM((EB, DW), jnp.float32),
        pltpu.VMEM((EB, DW), jnp.float32),
        pltpu.VMEM((EB, DW), jnp.float32),
        pltpu.VMEM((EB, DW), jnp.float32),
        pltpu.VMEM((N,), jnp.float32),
        pltpu.VMEM((N,), jnp.float32),
        pltpu.VMEM((128,), jnp.float32),
        pltpu.SemaphoreType.DMA,
        pltpu.SemaphoreType.DMA,
    ],
)
def _edge_kernel(src_tab, dst_tab, eidx, att_re, den_out, t_out,
                 src_idx, dst_idx, bs0, bs1, bd0, bd1, acc_d, acc_t,
                 att_v, sem0, sem1):
    cid = lax.axis_index("c")
    sid = lax.axis_index("s")
    wid = sid * 2 + cid
    ebase = wid * EPT

    pltpu.sync_copy(eidx.at[0, pl.ds(ebase, EPT)], src_idx)
    pltpu.sync_copy(eidx.at[1, pl.ds(ebase, EPT)], dst_idx)
    pltpu.sync_copy(att_re, att_v)

    def _zero(k, carry):
        acc_d[pl.ds(k * 16, 16)] = jnp.zeros((16,), jnp.float32)
        acc_t[pl.ds(k * 16, 16)] = jnp.zeros((16,), jnp.float32)
        return carry

    lax.fori_loop(0, N // 16, _zero, 0)

    iota16 = lax.iota(jnp.int32, 16)
    att_rot = [att_v[pl.ds(j * 16, 16)] for j in range(H)]
    col8 = [jnp.bitwise_and(iota16 + j, 7) for j in range(H)]
    col20 = []
    for j in range(C):
        cj = iota16 + j
        col20.append(jnp.where(cj >= C, cj - C, cj) + H)

    def fire(c, bs, bd, sem):
        pltpu.async_copy(src_tab.at[src_idx.at[pl.ds(c * EB, EB)]], bs, sem)
        pltpu.async_copy(dst_tab.at[dst_idx.at[pl.ds(c * EB, EB)]], bd, sem)

    def drain(bs, bd, sem):
        pltpu.make_async_copy(src_tab.at[pl.ds(0, EB)], bs, sem).wait()
        pltpu.make_async_copy(dst_tab.at[pl.ds(0, EB)], bd, sem).wait()

    def compute(c, bs, bd):
        for g in range(GPC):
            rows = iota16 + (g * 16)
            dstv = dst_idx[pl.ds(c * EB + g * 16, 16)]
            sa = jnp.zeros((16,), jnp.float32)
            for j in range(H):
                av = plsc.load_gather(bs, [rows, col8[j]])
                bv = plsc.load_gather(bd, [rows, col8[j]])
                z = av + bv
                z = jnp.where(z >= 0.0, z, 0.2 * z)
                sa = sa + att_rot[j] * z
            qc = jnp.zeros((16,), jnp.float32)
            for j in range(C):
                av = plsc.load_gather(bs, [rows, col20[j]])
                bv = plsc.load_gather(bd, [rows, col20[j]])
                qc = qc + av * bv
            wv = jnp.exp(sa)
            plsc.addupdate_scatter(acc_d, [dstv], wv)
            plsc.addupdate_scatter(acc_t, [dstv], wv * qc)

    fire(0, bs0, bd0, sem0)

    def body(cc, carry):
        c0 = cc * 2
        drain(bs0, bd0, sem0)
        fire(c0 + 1, bs1, bd1, sem1)
        compute(c0, bs0, bd0)
        drain(bs1, bd1, sem1)
        fire(c0 + 2, bs0, bd0, sem0)
        compute(c0 + 1, bs1, bd1)
        return carry

    lax.fori_loop(0, (NCHUNK - 1) // 2, body, 0)

    drain(bs0, bd0, sem0)
    compute(NCHUNK - 1, bs0, bd0)

    pltpu.sync_copy(acc_d, den_out.at[wid])
    pltpu.sync_copy(acc_t, t_out.at[wid])


def _fin_body(d_ref, t_ref, ce_ref):
    d = jnp.sum(d_ref[...], axis=0)
    t = jnp.sum(t_ref[...], axis=0)
    safe = jnp.where(d > 0.0, d, 1.0)
    ce = -jnp.sum(jnp.where(d > 0.0, t / safe, 0.0)) * (1.0 / N)
    ce_ref[...] = jnp.reshape(ce, (1, 1))


_fin_call = pl.pallas_call(
    _fin_body,
    in_specs=[
        pl.BlockSpec((NW, N), lambda: (0, 0)),
        pl.BlockSpec((NW, N), lambda: (0, 0)),
    ],
    out_specs=pl.BlockSpec((1, 1), lambda: (0, 0)),
    out_shape=jax.ShapeDtypeStruct((1, 1), jnp.float32),
)


def kernel(X, Mu, Var, W, S, lin_l_w, lin_l_b, lin_r_w, lin_r_b, att, edge_index):
    p, src_tab, dst_tab, ll = _dense_call(
        X, Mu, Var, W, S, lin_l_w, lin_r_w,
        lin_l_b.reshape(1, H), lin_r_b.reshape(1, H))
    r = jnp.arange(16)
    att_re = jnp.concatenate([att[(j + r) % H] for j in range(H)])
    den, t = _edge_kernel(src_tab, dst_tab, edge_index, att_re)
    ce = _fin_call(den, t)
    return (ll[0, 0], ce[0, 0], p)
